# baseline (speedup 1.0000x reference)
import jax
import jax.numpy as jnp
import numpy as np
from jax.experimental import pallas as pl

_EPS = float(np.finfo(float).eps)


def _tc_body(x_ref, g_ref, o_ref):
    e_total = x_ref.shape[0]
    acc = jnp.exp(x_ref[0]) * g_ref[0][None, :]
    for e in range(1, e_total):
        acc = acc + jnp.exp(x_ref[e]) * g_ref[e][None, :]
    o_ref[...] = jnp.log(jnp.where(acc == 0.0, _EPS, acc))


def kernel(xs, gates):
    E, B, P, C = xs.shape
    PC = P * C
    xs_t = jnp.transpose(xs, (0, 2, 3, 1)).reshape(E, PC, B)
    g_t = gates.T
    PCb = 768

    out_t = pl.pallas_call(
        _tc_body,
        grid=(PC // PCb,),
        in_specs=[
            pl.BlockSpec((E, PCb, B), lambda i: (0, i, 0)),
            pl.BlockSpec((E, B), lambda i: (0, 0)),
        ],
        out_specs=pl.BlockSpec((PCb, B), lambda i: (i, 0)),
        out_shape=jax.ShapeDtypeStruct((PC, B), jnp.float32),
    )(xs_t, g_t)
    return jnp.transpose(out_t.reshape(P, C, B), (2, 0, 1))


# speedup vs baseline: 1.0117x; 1.0117x over previous
import jax
import jax.numpy as jnp
import numpy as np
from jax.experimental import pallas as pl

_EPS = float(np.finfo(float).eps)


def _tc_body(x_ref, g_ref, o_ref):
    e_total = x_ref.shape[0]
    acc = jnp.exp(x_ref[0]) * g_ref[0][None, :]
    for e in range(1, e_total):
        acc = acc + jnp.exp(x_ref[e]) * g_ref[e][None, :]
    o_ref[...] = jnp.log(jnp.where(acc == 0.0, _EPS, acc))


def kernel(xs, gates):
    E, B, P, C = xs.shape
    PC = P * C
    xs_t = jnp.transpose(xs, (0, 2, 3, 1)).reshape(E, PC, B)
    g_t = gates.T
    PCb = 192

    out_t = pl.pallas_call(
        _tc_body,
        grid=(PC // PCb,),
        in_specs=[
            pl.BlockSpec((E, PCb, B), lambda i: (0, i, 0)),
            pl.BlockSpec((E, B), lambda i: (0, 0)),
        ],
        out_specs=pl.BlockSpec((PCb, B), lambda i: (i, 0)),
        out_shape=jax.ShapeDtypeStruct((PC, B), jnp.float32),
    )(xs_t, g_t)
    return jnp.transpose(out_t.reshape(P, C, B), (2, 0, 1))


# speedup vs baseline: 1.0391x; 1.0271x over previous
import jax
import jax.numpy as jnp
import numpy as np
from jax.experimental import pallas as pl

_EPS = float(np.finfo(float).eps)


def _tc_body(x_ref, g_ref, o_ref):
    e_total = x_ref.shape[0]
    acc = jnp.exp(x_ref[0]) * g_ref[0][None, :]
    for e in range(1, e_total):
        acc = acc + jnp.exp(x_ref[e]) * g_ref[e][None, :]
    o_ref[...] = jnp.log(jnp.where(acc == 0.0, _EPS, acc))


def kernel(xs, gates):
    E, B, P, C = xs.shape
    PC = P * C
    xs_t = jnp.transpose(xs, (0, 2, 3, 1)).reshape(E, PC, B)
    g_t = gates.T
    PCb = 384

    out_t = pl.pallas_call(
        _tc_body,
        grid=(PC // PCb,),
        in_specs=[
            pl.BlockSpec((E, PCb, B), lambda i: (0, i, 0)),
            pl.BlockSpec((E, B), lambda i: (0, 0)),
        ],
        out_specs=pl.BlockSpec((PCb, B), lambda i: (i, 0)),
        out_shape=jax.ShapeDtypeStruct((PC, B), jnp.float32),
    )(xs_t, g_t)
    return jnp.transpose(out_t.reshape(P, C, B), (2, 0, 1))
